# Initial kernel scaffold; baseline (speedup 1.0000x reference)
import jax
import jax.numpy as jnp
from jax.experimental import pallas as pl


def kernel(x, edge_index, edge_attr, W_node, W_edge, bias):
    raise NotImplementedError("write your pallas kernel here")


# speedup vs baseline: 7.9596x; 7.9596x over previous
import functools

import jax
import jax.numpy as jnp
from jax import lax
from jax.experimental import pallas as pl
from jax.experimental.pallas import tpu as pltpu
from jax.experimental.pallas import tpu_sc as plsc

N_NODES = 10000
N_EDGES = 320000
D_IN = 128
D_OUT = 128
D_EDGE = 16

NC = 2
NS = 16
NW = NC * NS
E_PER_W = N_EDGES // NW
CHUNK = 80
N_CHUNKS = E_PER_W // CHUNK
N_PAD = 10112
ROWS_PER_TILE = N_PAD // NS

_sc_mesh = plsc.VectorSubcoreMesh(core_axis_name="c", subcore_axis_name="s")


@functools.partial(
    pl.kernel,
    out_type=jax.ShapeDtypeStruct((NC * N_PAD, D_OUT), jnp.float32),
    mesh=_sc_mesh,
    scratch_types=[
        pltpu.VMEM((CHUNK,), jnp.int32),
        pltpu.VMEM((CHUNK, D_OUT), jnp.float32),
        pltpu.VMEM_SHARED((N_PAD, D_OUT), jnp.float32),
    ],
)
def _deg_kernel(col_hbm, ones_hbm, z_hbm, deg_hbm, colv, onesv, deg_sh):
    core = lax.axis_index("c")
    sid = lax.axis_index("s")
    wid = sid * NC + core
    r0 = sid * ROWS_PER_TILE

    pltpu.sync_copy(z_hbm, deg_sh.at[pl.ds(r0, ROWS_PER_TILE)])
    pltpu.sync_copy(ones_hbm, onesv)
    plsc.subcore_barrier()

    def body(k, _):
        base = wid * E_PER_W + k * CHUNK
        pltpu.sync_copy(col_hbm.at[pl.ds(base, CHUNK)], colv)
        pltpu.sync_copy(onesv, deg_sh.at[colv], add=True)
        return ()

    lax.fori_loop(0, N_CHUNKS, body, ())
    plsc.subcore_barrier()
    pltpu.sync_copy(deg_sh.at[pl.ds(r0, ROWS_PER_TILE)],
                    deg_hbm.at[pl.ds(core * N_PAD + r0, ROWS_PER_TILE)])


def _scale_matmul_body(x_ref, w_ref, deg_ref, y_ref, dinv_ref):
    xt = jnp.dot(x_ref[...], w_ref[...], preferred_element_type=jnp.float32)
    d = deg_ref[:, 0:1] + deg_ref[:, 1:2] + 1.0
    dv = lax.rsqrt(d)
    dinv_ref[...] = dv
    y_ref[...] = xt * dv


@functools.partial(
    pl.kernel,
    out_type=jax.ShapeDtypeStruct((NC * N_PAD, D_OUT), jnp.float32),
    mesh=_sc_mesh,
    scratch_types=[
        pltpu.VMEM((CHUNK,), jnp.int32),
        pltpu.VMEM((CHUNK,), jnp.int32),
        pltpu.VMEM((CHUNK, D_OUT), jnp.float32),
        pltpu.SemaphoreType.DMA,
        pltpu.VMEM_SHARED((N_PAD, D_OUT), jnp.float32),
    ],
)
def _node_agg_kernel(row_hbm, col_hbm, y_hbm, zp_hbm, p_hbm,
                     rowv, colv, gbuf, sem, p_sh):
    core = lax.axis_index("c")
    sid = lax.axis_index("s")
    wid = sid * NC + core
    r0 = sid * ROWS_PER_TILE

    pltpu.sync_copy(zp_hbm, p_sh.at[pl.ds(r0, ROWS_PER_TILE)])
    plsc.subcore_barrier()

    def body(k, _):
        base = wid * E_PER_W + k * CHUNK
        pltpu.sync_copy(row_hbm.at[pl.ds(base, CHUNK)], rowv)
        pltpu.sync_copy(col_hbm.at[pl.ds(base, CHUNK)], colv)
        pltpu.async_copy(y_hbm.at[rowv], gbuf, sem).wait()
        pltpu.sync_copy(gbuf, p_sh.at[colv], add=True)
        return ()

    lax.fori_loop(0, N_CHUNKS, body, ())
    plsc.subcore_barrier()
    pltpu.sync_copy(p_sh.at[pl.ds(r0, ROWS_PER_TILE)],
                    p_hbm.at[pl.ds(core * N_PAD + r0, ROWS_PER_TILE)])


@functools.partial(
    pl.kernel,
    out_type=jax.ShapeDtypeStruct((NC * N_PAD, D_OUT), jnp.float32),
    mesh=_sc_mesh,
    scratch_types=[
        pltpu.VMEM((CHUNK,), jnp.int32),
        pltpu.VMEM((CHUNK,), jnp.int32),
        pltpu.VMEM((CHUNK, D_EDGE), jnp.float32),
        pltpu.VMEM((N_NODES + 16,), jnp.float32),
        pltpu.VMEM((CHUNK, D_OUT), jnp.float32),
        pltpu.VMEM_SHARED((N_PAD, D_OUT), jnp.float32),
    ],
)
def _edge_agg_kernel(row_hbm, col_hbm, ea_hbm, dinv_hbm, zq_hbm, q_hbm,
                     rowv, colv, eabuf, dloc, msgbuf, q_sh):
    core = lax.axis_index("c")
    sid = lax.axis_index("s")
    wid = sid * NC + core
    r0 = sid * ROWS_PER_TILE

    pltpu.sync_copy(zq_hbm, q_sh.at[pl.ds(r0, ROWS_PER_TILE)])
    pltpu.sync_copy(zq_hbm.at[pl.ds(0, CHUNK)], msgbuf)
    pltpu.sync_copy(dinv_hbm, dloc.at[pl.ds(0, N_NODES)])
    plsc.subcore_barrier()

    def body(k, _):
        base = wid * E_PER_W + k * CHUNK
        pltpu.sync_copy(row_hbm.at[pl.ds(base, CHUNK)], rowv)
        pltpu.sync_copy(col_hbm.at[pl.ds(base, CHUNK)], colv)
        pltpu.sync_copy(ea_hbm.at[pl.ds(base, CHUNK)], eabuf)

        def scale(j, _):
            rv = rowv[pl.ds(j * 16, 16)]
            for i in range(16):
                e = j * 16 + i
                dv = dloc[pl.ds(rv[i], 16)]
                msgbuf[e, pl.ds(0, D_EDGE)] = eabuf[e, :] * dv[0]
            return ()

        lax.fori_loop(0, CHUNK // 16, scale, ())
        pltpu.sync_copy(msgbuf, q_sh.at[colv], add=True)
        return ()

    lax.fori_loop(0, N_CHUNKS, body, ())
    plsc.subcore_barrier()
    pltpu.sync_copy(q_sh.at[pl.ds(r0, ROWS_PER_TILE)],
                    q_hbm.at[pl.ds(core * N_PAD + r0, ROWS_PER_TILE)])


def _combine_body(p_ref, q_ref, y_ref, dinv_ref, we_ref, b_ref, out_ref):
    q = q_ref[0][:, :D_EDGE] + q_ref[1][:, :D_EDGE]
    eat = jnp.dot(q, we_ref[...], preferred_element_type=jnp.float32)
    s = p_ref[0] + p_ref[1] + y_ref[...] + eat
    out_ref[...] = s * dinv_ref[...] + b_ref[...]


BLK = 400
N_BLKS = N_NODES // BLK


def kernel(x, edge_index, edge_attr, W_node, W_edge, bias):
    row = edge_index[0].astype(jnp.int32)
    col = edge_index[1].astype(jnp.int32)
    ea = edge_attr.astype(jnp.float32)

    ones_c = jnp.ones((CHUNK, D_OUT), jnp.float32)
    z_rows = jnp.zeros((ROWS_PER_TILE, D_OUT), jnp.float32)

    deg2 = _deg_kernel(col, ones_c, z_rows)
    deg2t = deg2.reshape(NC, N_PAD, D_OUT)[:, :N_NODES, 0].T

    y, dinv = pl.pallas_call(
        _scale_matmul_body,
        grid=(N_BLKS,),
        in_specs=[
            pl.BlockSpec((BLK, D_IN), lambda i: (i, 0)),
            pl.BlockSpec((D_IN, D_OUT), lambda i: (0, 0)),
            pl.BlockSpec((BLK, 2), lambda i: (i, 0)),
        ],
        out_specs=[
            pl.BlockSpec((BLK, D_OUT), lambda i: (i, 0)),
            pl.BlockSpec((BLK, 1), lambda i: (i, 0)),
        ],
        out_shape=[
            jax.ShapeDtypeStruct((N_NODES, D_OUT), jnp.float32),
            jax.ShapeDtypeStruct((N_NODES, 1), jnp.float32),
        ],
    )(x, W_node, deg2t)

    p2 = _node_agg_kernel(row, col, y, z_rows)
    q2 = _edge_agg_kernel(row, col, ea, dinv.reshape(N_NODES), z_rows)
    p2 = p2.reshape(NC, N_PAD, D_OUT)[:, :N_NODES]
    q2 = q2.reshape(NC, N_PAD, D_OUT)[:, :N_NODES]

    out = pl.pallas_call(
        _combine_body,
        grid=(N_BLKS,),
        in_specs=[
            pl.BlockSpec((NC, BLK, D_OUT), lambda i: (0, i, 0)),
            pl.BlockSpec((NC, BLK, D_OUT), lambda i: (0, i, 0)),
            pl.BlockSpec((BLK, D_OUT), lambda i: (i, 0)),
            pl.BlockSpec((BLK, 1), lambda i: (i, 0)),
            pl.BlockSpec((D_EDGE, D_OUT), lambda i: (0, 0)),
            pl.BlockSpec((1, D_OUT), lambda i: (0, 0)),
        ],
        out_specs=pl.BlockSpec((BLK, D_OUT), lambda i: (i, 0)),
        out_shape=jax.ShapeDtypeStruct((N_NODES, D_OUT), jnp.float32),
    )(p2, q2, y, dinv, W_edge.astype(jnp.float32), bias.reshape(1, D_OUT))
    return out


# speedup vs baseline: 11.3522x; 1.4262x over previous
import functools

import jax
import jax.numpy as jnp
from jax import lax
from jax.experimental import pallas as pl
from jax.experimental.pallas import tpu as pltpu
from jax.experimental.pallas import tpu_sc as plsc

N_NODES = 10000
N_EDGES = 320000
D_IN = 128
D_OUT = 128
D_EDGE = 16

NC = 2
NS = 16
NW = NC * NS
E_PER_W = N_EDGES // NW
CHUNK = 80
N_CHUNKS = E_PER_W // CHUNK
N_PAD = 10112
ROWS_PER_TILE = N_PAD // NS

_sc_mesh = plsc.VectorSubcoreMesh(core_axis_name="c", subcore_axis_name="s")


@functools.partial(
    pl.kernel,
    out_type=jax.ShapeDtypeStruct((NC * N_PAD, D_OUT), jnp.float32),
    mesh=_sc_mesh,
    scratch_types=[
        pltpu.VMEM((2, CHUNK), jnp.int32),
        pltpu.VMEM((CHUNK, D_OUT), jnp.float32),
        pltpu.SemaphoreType.DMA,
        pltpu.SemaphoreType.DMA,
        pltpu.VMEM_SHARED((N_PAD, D_OUT), jnp.float32),
    ],
)
def _deg_kernel(col_hbm, ones_hbm, z_hbm, deg_hbm, colv2, onesv, sem_i,
                sem_s, deg_sh):
    core = lax.axis_index("c")
    sid = lax.axis_index("s")
    wid = sid * NC + core
    r0 = sid * ROWS_PER_TILE

    pltpu.sync_copy(z_hbm, deg_sh.at[pl.ds(r0, ROWS_PER_TILE)])
    pltpu.sync_copy(ones_hbm, onesv)
    plsc.subcore_barrier()

    base_w = wid * E_PER_W
    pltpu.sync_copy(col_hbm.at[pl.ds(base_w, CHUNK)], colv2.at[0])
    pltpu.sync_copy(onesv, deg_sh.at[colv2.at[0]], add=True)

    def body(j, _):
        base = base_w + (1 + 2 * j) * CHUNK
        dca = pltpu.async_copy(col_hbm.at[pl.ds(base, CHUNK)],
                               colv2.at[0], sem_i)
        dcb = pltpu.async_copy(col_hbm.at[pl.ds(base + CHUNK, CHUNK)],
                               colv2.at[1], sem_i)
        dca.wait()
        dcb.wait()
        d0 = pltpu.async_copy(onesv, deg_sh.at[colv2.at[0]], sem_s, add=True)
        d0.wait()
        d1 = pltpu.async_copy(onesv, deg_sh.at[colv2.at[1]], sem_s, add=True)
        d1.wait()
        return ()

    lax.fori_loop(0, (N_CHUNKS - 1) // 2, body, ())
    plsc.subcore_barrier()
    pltpu.sync_copy(deg_sh.at[pl.ds(r0, ROWS_PER_TILE)],
                    deg_hbm.at[pl.ds(core * N_PAD + r0, ROWS_PER_TILE)])


def _scale_matmul_body(x_ref, w_ref, deg_ref, y_ref, dinv_ref):
    xt = jnp.dot(x_ref[...], w_ref[...], preferred_element_type=jnp.float32)
    d = deg_ref[:, 0:1] + deg_ref[:, 1:2] + 1.0
    dv = lax.rsqrt(d)
    dinv_ref[...] = dv
    y_ref[...] = xt * dv


@functools.partial(
    pl.kernel,
    out_type=jax.ShapeDtypeStruct((NC * N_PAD, D_OUT), jnp.float32),
    mesh=_sc_mesh,
    scratch_types=[
        pltpu.VMEM((2 * CHUNK,), jnp.int32),
        pltpu.VMEM((2, CHUNK), jnp.int32),
        pltpu.VMEM((CHUNK, D_OUT), jnp.float32),
        pltpu.VMEM((CHUNK, D_OUT), jnp.float32),
        pltpu.SemaphoreType.DMA,
        pltpu.SemaphoreType.DMA,
        pltpu.VMEM_SHARED((N_PAD, D_OUT), jnp.float32),
    ],
)
def _node_agg_kernel(row_hbm, col_hbm, y_hbm, zp_hbm, p_hbm,
                     rowv2, colv2, g0, g1, sem_i, sem_g, p_sh):
    core = lax.axis_index("c")
    sid = lax.axis_index("s")
    wid = sid * NC + core
    r0 = sid * ROWS_PER_TILE

    pltpu.sync_copy(zp_hbm, p_sh.at[pl.ds(r0, ROWS_PER_TILE)])
    plsc.subcore_barrier()

    base_w = wid * E_PER_W

    pltpu.sync_copy(row_hbm.at[pl.ds(base_w, CHUNK)],
                    rowv2.at[pl.ds(0, CHUNK)])
    pltpu.sync_copy(col_hbm.at[pl.ds(base_w, CHUNK)], colv2.at[0])
    pltpu.async_copy(y_hbm.at[rowv2.at[pl.ds(0, CHUNK)]], g0, sem_g).wait()
    pltpu.sync_copy(g0, p_sh.at[colv2.at[0]], add=True)

    def body(j, _):
        base = base_w + (1 + 2 * j) * CHUNK
        di = pltpu.async_copy(row_hbm.at[pl.ds(base, 2 * CHUNK)], rowv2,
                              sem_i)
        dca = pltpu.async_copy(col_hbm.at[pl.ds(base, CHUNK)],
                               colv2.at[0], sem_i)
        dcb = pltpu.async_copy(col_hbm.at[pl.ds(base + CHUNK, CHUNK)],
                               colv2.at[1], sem_i)
        di.wait()
        dca.wait()
        dcb.wait()
        dg0 = pltpu.async_copy(y_hbm.at[rowv2.at[pl.ds(0, CHUNK)]], g0,
                               sem_g)
        dg1 = pltpu.async_copy(y_hbm.at[rowv2.at[pl.ds(CHUNK, CHUNK)]], g1,
                               sem_g)
        dg0.wait()
        pltpu.sync_copy(g0, p_sh.at[colv2.at[0]], add=True)
        dg1.wait()
        pltpu.sync_copy(g1, p_sh.at[colv2.at[1]], add=True)
        return ()

    lax.fori_loop(0, (N_CHUNKS - 1) // 2, body, ())
    plsc.subcore_barrier()
    pltpu.sync_copy(p_sh.at[pl.ds(r0, ROWS_PER_TILE)],
                    p_hbm.at[pl.ds(core * N_PAD + r0, ROWS_PER_TILE)])


@functools.partial(
    pl.kernel,
    out_type=jax.ShapeDtypeStruct((NC * N_PAD, D_OUT), jnp.float32),
    mesh=_sc_mesh,
    scratch_types=[
        pltpu.VMEM((2 * CHUNK,), jnp.int32),
        pltpu.VMEM((2, CHUNK), jnp.int32),
        pltpu.VMEM((CHUNK, D_EDGE), jnp.float32),
        pltpu.VMEM((N_NODES + 16,), jnp.float32),
        pltpu.VMEM((CHUNK, D_OUT), jnp.float32),
        pltpu.SemaphoreType.DMA,
        pltpu.SemaphoreType.DMA,
        pltpu.VMEM_SHARED((N_PAD, D_OUT), jnp.float32),
    ],
)
def _edge_agg_kernel(row_hbm, col_hbm, ea_hbm, dinv_hbm, zq_hbm, q_hbm,
                     rowv2, colv2, eabuf, dloc, m0, sem_i, sem_s, q_sh):
    core = lax.axis_index("c")
    sid = lax.axis_index("s")
    wid = sid * NC + core
    r0 = sid * ROWS_PER_TILE

    pltpu.sync_copy(zq_hbm, q_sh.at[pl.ds(r0, ROWS_PER_TILE)])
    pltpu.sync_copy(zq_hbm.at[pl.ds(0, CHUNK)], m0)
    pltpu.sync_copy(dinv_hbm, dloc.at[pl.ds(0, N_NODES)])
    plsc.subcore_barrier()

    base_w = wid * E_PER_W

    def scale(off):
        def step(j, _):
            rv = rowv2[pl.ds(off + j * 16, 16)]
            for i in range(16):
                e = j * 16 + i
                dv = dloc[pl.ds(rv[i], 16)]
                m0[e, pl.ds(0, D_EDGE)] = eabuf[e, :] * dv[0]
            return ()

        lax.fori_loop(0, CHUNK // 16, step, ())

    pltpu.sync_copy(row_hbm.at[pl.ds(base_w, CHUNK)],
                    rowv2.at[pl.ds(0, CHUNK)])
    pltpu.sync_copy(col_hbm.at[pl.ds(base_w, CHUNK)], colv2.at[0])
    pltpu.sync_copy(ea_hbm.at[pl.ds(base_w, CHUNK)], eabuf)
    scale(0)
    pltpu.sync_copy(m0, q_sh.at[colv2.at[0]], add=True)

    def body(j, _):
        base = base_w + (1 + 2 * j) * CHUNK
        di = pltpu.async_copy(row_hbm.at[pl.ds(base, 2 * CHUNK)], rowv2,
                              sem_i)
        dca = pltpu.async_copy(col_hbm.at[pl.ds(base, CHUNK)],
                               colv2.at[0], sem_i)
        dcb = pltpu.async_copy(col_hbm.at[pl.ds(base + CHUNK, CHUNK)],
                               colv2.at[1], sem_i)
        de0 = pltpu.async_copy(ea_hbm.at[pl.ds(base, CHUNK)], eabuf,
                               sem_i)
        di.wait()
        de0.wait()
        dca.wait()
        dcb.wait()
        scale(0)
        d0 = pltpu.async_copy(m0, q_sh.at[colv2.at[0]], sem_s, add=True)
        de1 = pltpu.async_copy(ea_hbm.at[pl.ds(base + CHUNK, CHUNK)],
                               eabuf, sem_i)
        d0.wait()
        de1.wait()
        scale(CHUNK)
        pltpu.sync_copy(m0, q_sh.at[colv2.at[1]], add=True)
        return ()

    lax.fori_loop(0, (N_CHUNKS - 1) // 2, body, ())
    plsc.subcore_barrier()
    pltpu.sync_copy(q_sh.at[pl.ds(r0, ROWS_PER_TILE)],
                    q_hbm.at[pl.ds(core * N_PAD + r0, ROWS_PER_TILE)])


def _combine_body(p_ref, q_ref, y_ref, dinv_ref, we_ref, b_ref, out_ref):
    q = q_ref[0][:, :D_EDGE] + q_ref[1][:, :D_EDGE]
    eat = jnp.dot(q, we_ref[...], preferred_element_type=jnp.float32)
    s = p_ref[0] + p_ref[1] + y_ref[...] + eat
    out_ref[...] = s * dinv_ref[...] + b_ref[...]


BLK = 400
N_BLKS = N_NODES // BLK


def kernel(x, edge_index, edge_attr, W_node, W_edge, bias):
    row = edge_index[0].astype(jnp.int32)
    col = edge_index[1].astype(jnp.int32)
    ea = edge_attr.astype(jnp.float32)

    ones_c = jnp.ones((CHUNK, D_OUT), jnp.float32)
    z_rows = jnp.zeros((ROWS_PER_TILE, D_OUT), jnp.float32)

    deg2 = _deg_kernel(col, ones_c, z_rows)
    deg2t = deg2.reshape(NC, N_PAD, D_OUT)[:, :N_NODES, 0].T

    y, dinv = pl.pallas_call(
        _scale_matmul_body,
        grid=(N_BLKS,),
        in_specs=[
            pl.BlockSpec((BLK, D_IN), lambda i: (i, 0)),
            pl.BlockSpec((D_IN, D_OUT), lambda i: (0, 0)),
            pl.BlockSpec((BLK, 2), lambda i: (i, 0)),
        ],
        out_specs=[
            pl.BlockSpec((BLK, D_OUT), lambda i: (i, 0)),
            pl.BlockSpec((BLK, 1), lambda i: (i, 0)),
        ],
        out_shape=[
            jax.ShapeDtypeStruct((N_NODES, D_OUT), jnp.float32),
            jax.ShapeDtypeStruct((N_NODES, 1), jnp.float32),
        ],
    )(x, W_node, deg2t)

    p2 = _node_agg_kernel(row, col, y, z_rows)
    q2 = _edge_agg_kernel(row, col, ea, dinv.reshape(N_NODES), z_rows)
    p2 = p2.reshape(NC, N_PAD, D_OUT)[:, :N_NODES]
    q2 = q2.reshape(NC, N_PAD, D_OUT)[:, :N_NODES]

    out = pl.pallas_call(
        _combine_body,
        grid=(N_BLKS,),
        in_specs=[
            pl.BlockSpec((NC, BLK, D_OUT), lambda i: (0, i, 0)),
            pl.BlockSpec((NC, BLK, D_OUT), lambda i: (0, i, 0)),
            pl.BlockSpec((BLK, D_OUT), lambda i: (i, 0)),
            pl.BlockSpec((BLK, 1), lambda i: (i, 0)),
            pl.BlockSpec((D_EDGE, D_OUT), lambda i: (0, 0)),
            pl.BlockSpec((1, D_OUT), lambda i: (0, 0)),
        ],
        out_specs=pl.BlockSpec((BLK, D_OUT), lambda i: (i, 0)),
        out_shape=jax.ShapeDtypeStruct((N_NODES, D_OUT), jnp.float32),
    )(p2, q2, y, dinv, W_edge.astype(jnp.float32), bias.reshape(1, D_OUT))
    return out


# speedup vs baseline: 12.6488x; 1.1142x over previous
import functools

import jax
import jax.numpy as jnp
from jax import lax
from jax.experimental import pallas as pl
from jax.experimental.pallas import tpu as pltpu
from jax.experimental.pallas import tpu_sc as plsc

N_NODES = 10000
N_EDGES = 320000
D_IN = 128
D_OUT = 128
D_EDGE = 16

NC = 2
NS = 16
NW = NC * NS
E_PER_W = N_EDGES // NW
CHUNK = 80
N_CHUNKS = E_PER_W // CHUNK
N_PAD = 10112
ROWS_PER_TILE = N_PAD // NS

_sc_mesh = plsc.VectorSubcoreMesh(core_axis_name="c", subcore_axis_name="s")


@functools.partial(
    pl.kernel,
    out_type=jax.ShapeDtypeStruct((NC * N_PAD, D_OUT), jnp.float32),
    mesh=_sc_mesh,
    scratch_types=[
        pltpu.VMEM((2, CHUNK), jnp.int32),
        pltpu.VMEM((CHUNK, D_OUT), jnp.float32),
        pltpu.SemaphoreType.DMA,
        pltpu.SemaphoreType.DMA,
        pltpu.VMEM_SHARED((N_PAD, D_OUT), jnp.float32),
    ],
)
def _deg_kernel(col_hbm, ones_hbm, z_hbm, deg_hbm, colv2, onesv, sem_i,
                sem_s, deg_sh):
    core = lax.axis_index("c")
    sid = lax.axis_index("s")
    wid = sid * NC + core
    r0 = sid * ROWS_PER_TILE

    pltpu.sync_copy(z_hbm, deg_sh.at[pl.ds(r0, ROWS_PER_TILE)])
    pltpu.sync_copy(ones_hbm, onesv)
    plsc.subcore_barrier()

    base_w = wid * E_PER_W
    pltpu.sync_copy(col_hbm.at[pl.ds(base_w, CHUNK)], colv2.at[0])
    pltpu.sync_copy(onesv, deg_sh.at[colv2.at[0]], add=True)

    def body(j, _):
        base = base_w + (1 + 2 * j) * CHUNK
        dca = pltpu.async_copy(col_hbm.at[pl.ds(base, CHUNK)],
                               colv2.at[0], sem_i)
        dcb = pltpu.async_copy(col_hbm.at[pl.ds(base + CHUNK, CHUNK)],
                               colv2.at[1], sem_i)
        dca.wait()
        dcb.wait()
        d0 = pltpu.async_copy(onesv, deg_sh.at[colv2.at[0]], sem_s, add=True)
        d0.wait()
        d1 = pltpu.async_copy(onesv, deg_sh.at[colv2.at[1]], sem_s, add=True)
        d1.wait()
        return ()

    lax.fori_loop(0, (N_CHUNKS - 1) // 2, body, ())
    plsc.subcore_barrier()
    pltpu.sync_copy(deg_sh.at[pl.ds(r0, ROWS_PER_TILE)],
                    deg_hbm.at[pl.ds(core * N_PAD + r0, ROWS_PER_TILE)])


def _scale_matmul_body(x_ref, w_ref, d0_ref, d1_ref, y_ref, dinv_ref):
    xt = jnp.dot(x_ref[...], w_ref[...], preferred_element_type=jnp.float32)
    d = d0_ref[:, 0:1] + d1_ref[:, 0:1] + 1.0
    dv = lax.rsqrt(d)
    dinv_ref[...] = dv
    y_ref[...] = xt * dv


@functools.partial(
    pl.kernel,
    out_type=jax.ShapeDtypeStruct((NC * N_PAD, D_OUT), jnp.float32),
    mesh=_sc_mesh,
    scratch_types=[
        pltpu.VMEM((2 * CHUNK,), jnp.int32),
        pltpu.VMEM((2, CHUNK), jnp.int32),
        pltpu.VMEM((CHUNK, D_OUT), jnp.float32),
        pltpu.VMEM((CHUNK, D_OUT), jnp.float32),
        pltpu.SemaphoreType.DMA,
        pltpu.SemaphoreType.DMA,
        pltpu.VMEM_SHARED((N_PAD, D_OUT), jnp.float32),
    ],
)
def _node_agg_kernel(row_hbm, col_hbm, y_hbm, zp_hbm, p_hbm,
                     rowv2, colv2, g0, g1, sem_i, sem_g, p_sh):
    core = lax.axis_index("c")
    sid = lax.axis_index("s")
    wid = sid * NC + core
    r0 = sid * ROWS_PER_TILE

    pltpu.sync_copy(zp_hbm, p_sh.at[pl.ds(r0, ROWS_PER_TILE)])
    plsc.subcore_barrier()

    base_w = wid * E_PER_W

    pltpu.sync_copy(row_hbm.at[pl.ds(base_w, CHUNK)],
                    rowv2.at[pl.ds(0, CHUNK)])
    pltpu.sync_copy(col_hbm.at[pl.ds(base_w, CHUNK)], colv2.at[0])
    pltpu.async_copy(y_hbm.at[rowv2.at[pl.ds(0, CHUNK)]], g0, sem_g).wait()
    pltpu.sync_copy(g0, p_sh.at[colv2.at[0]], add=True)

    def body(j, _):
        base = base_w + (1 + 2 * j) * CHUNK
        di = pltpu.async_copy(row_hbm.at[pl.ds(base, 2 * CHUNK)], rowv2,
                              sem_i)
        dca = pltpu.async_copy(col_hbm.at[pl.ds(base, CHUNK)],
                               colv2.at[0], sem_i)
        dcb = pltpu.async_copy(col_hbm.at[pl.ds(base + CHUNK, CHUNK)],
                               colv2.at[1], sem_i)
        di.wait()
        dca.wait()
        dcb.wait()
        dg0 = pltpu.async_copy(y_hbm.at[rowv2.at[pl.ds(0, CHUNK)]], g0,
                               sem_g)
        dg1 = pltpu.async_copy(y_hbm.at[rowv2.at[pl.ds(CHUNK, CHUNK)]], g1,
                               sem_g)
        dg0.wait()
        pltpu.sync_copy(g0, p_sh.at[colv2.at[0]], add=True)
        dg1.wait()
        pltpu.sync_copy(g1, p_sh.at[colv2.at[1]], add=True)
        return ()

    lax.fori_loop(0, (N_CHUNKS - 1) // 2, body, ())
    plsc.subcore_barrier()
    pltpu.sync_copy(p_sh.at[pl.ds(r0, ROWS_PER_TILE)],
                    p_hbm.at[pl.ds(core * N_PAD + r0, ROWS_PER_TILE)])


@functools.partial(
    pl.kernel,
    out_type=jax.ShapeDtypeStruct((NC * N_PAD, D_OUT), jnp.float32),
    mesh=_sc_mesh,
    scratch_types=[
        pltpu.VMEM((2 * CHUNK,), jnp.int32),
        pltpu.VMEM((2, CHUNK), jnp.int32),
        pltpu.VMEM((CHUNK, D_EDGE), jnp.float32),
        pltpu.VMEM((N_NODES + 16,), jnp.float32),
        pltpu.VMEM((CHUNK, D_OUT), jnp.float32),
        pltpu.SemaphoreType.DMA,
        pltpu.SemaphoreType.DMA,
        pltpu.VMEM_SHARED((N_PAD, D_OUT), jnp.float32),
    ],
)
def _edge_agg_kernel(row_hbm, col_hbm, ea_hbm, dinv_hbm, zq_hbm, q_hbm,
                     rowv2, colv2, eabuf, dloc, m0, sem_i, sem_s, q_sh):
    core = lax.axis_index("c")
    sid = lax.axis_index("s")
    wid = sid * NC + core
    r0 = sid * ROWS_PER_TILE

    pltpu.sync_copy(zq_hbm, q_sh.at[pl.ds(r0, ROWS_PER_TILE)])
    pltpu.sync_copy(zq_hbm.at[pl.ds(0, CHUNK)], m0)
    pltpu.sync_copy(dinv_hbm, dloc.at[pl.ds(0, N_NODES)])
    plsc.subcore_barrier()

    base_w = wid * E_PER_W

    def scale(off):
        def step(j, _):
            rv = rowv2[pl.ds(off + j * 16, 16)]
            for i in range(16):
                e = j * 16 + i
                dv = dloc[pl.ds(rv[i], 16)]
                m0[e, pl.ds(0, D_EDGE)] = eabuf[e, :] * dv[0]
            return ()

        lax.fori_loop(0, CHUNK // 16, step, ())

    pltpu.sync_copy(row_hbm.at[pl.ds(base_w, CHUNK)],
                    rowv2.at[pl.ds(0, CHUNK)])
    pltpu.sync_copy(col_hbm.at[pl.ds(base_w, CHUNK)], colv2.at[0])
    pltpu.sync_copy(ea_hbm.at[pl.ds(base_w, CHUNK)], eabuf)
    scale(0)
    pltpu.sync_copy(m0, q_sh.at[colv2.at[0]], add=True)

    def body(j, _):
        base = base_w + (1 + 2 * j) * CHUNK
        di = pltpu.async_copy(row_hbm.at[pl.ds(base, 2 * CHUNK)], rowv2,
                              sem_i)
        dca = pltpu.async_copy(col_hbm.at[pl.ds(base, CHUNK)],
                               colv2.at[0], sem_i)
        dcb = pltpu.async_copy(col_hbm.at[pl.ds(base + CHUNK, CHUNK)],
                               colv2.at[1], sem_i)
        de0 = pltpu.async_copy(ea_hbm.at[pl.ds(base, CHUNK)], eabuf,
                               sem_i)
        di.wait()
        de0.wait()
        dca.wait()
        dcb.wait()
        scale(0)
        d0 = pltpu.async_copy(m0, q_sh.at[colv2.at[0]], sem_s, add=True)
        de1 = pltpu.async_copy(ea_hbm.at[pl.ds(base + CHUNK, CHUNK)],
                               eabuf, sem_i)
        d0.wait()
        de1.wait()
        scale(CHUNK)
        pltpu.sync_copy(m0, q_sh.at[colv2.at[1]], add=True)
        return ()

    lax.fori_loop(0, (N_CHUNKS - 1) // 2, body, ())
    plsc.subcore_barrier()
    pltpu.sync_copy(q_sh.at[pl.ds(r0, ROWS_PER_TILE)],
                    q_hbm.at[pl.ds(core * N_PAD + r0, ROWS_PER_TILE)])


def _combine_body(p_ref, q_ref, y_ref, dinv_ref, we_ref, b_ref, out_ref):
    q = q_ref[0][:, :D_EDGE] + q_ref[1][:, :D_EDGE]
    eat = jnp.dot(q, we_ref[...], preferred_element_type=jnp.float32)
    s = p_ref[0] + p_ref[1] + y_ref[...] + eat
    out_ref[...] = s * dinv_ref[...] + b_ref[...]


BLK = 400
N_BLKS = N_NODES // BLK


def kernel(x, edge_index, edge_attr, W_node, W_edge, bias):
    row = edge_index[0].astype(jnp.int32)
    col = edge_index[1].astype(jnp.int32)
    ea = edge_attr.astype(jnp.float32)

    ones_c = jnp.ones((CHUNK, D_OUT), jnp.float32)
    z_rows = jnp.zeros((ROWS_PER_TILE, D_OUT), jnp.float32)

    deg2 = _deg_kernel(col, ones_c, z_rows)
    deg2 = deg2.reshape(NC, N_PAD, D_OUT)

    y, dinv = pl.pallas_call(
        _scale_matmul_body,
        grid=(N_BLKS,),
        in_specs=[
            pl.BlockSpec((BLK, D_IN), lambda i: (i, 0)),
            pl.BlockSpec((D_IN, D_OUT), lambda i: (0, 0)),
            pl.BlockSpec((BLK, D_OUT), lambda i: (i, 0)),
            pl.BlockSpec((BLK, D_OUT), lambda i: (i, 0)),
        ],
        out_specs=[
            pl.BlockSpec((BLK, D_OUT), lambda i: (i, 0)),
            pl.BlockSpec((BLK, 1), lambda i: (i, 0)),
        ],
        out_shape=[
            jax.ShapeDtypeStruct((N_NODES, D_OUT), jnp.float32),
            jax.ShapeDtypeStruct((N_NODES, 1), jnp.float32),
        ],
    )(x, W_node, deg2[0], deg2[1])

    p2 = _node_agg_kernel(row, col, y, z_rows).reshape(NC, N_PAD, D_OUT)
    q2 = _edge_agg_kernel(row, col, ea, dinv.reshape(N_NODES),
                          z_rows).reshape(NC, N_PAD, D_OUT)

    out = pl.pallas_call(
        _combine_body,
        grid=(N_BLKS,),
        in_specs=[
            pl.BlockSpec((NC, BLK, D_OUT), lambda i: (0, i, 0)),
            pl.BlockSpec((NC, BLK, D_OUT), lambda i: (0, i, 0)),
            pl.BlockSpec((BLK, D_OUT), lambda i: (i, 0)),
            pl.BlockSpec((BLK, 1), lambda i: (i, 0)),
            pl.BlockSpec((D_EDGE, D_OUT), lambda i: (0, 0)),
            pl.BlockSpec((1, D_OUT), lambda i: (0, 0)),
        ],
        out_specs=pl.BlockSpec((BLK, D_OUT), lambda i: (i, 0)),
        out_shape=jax.ShapeDtypeStruct((N_NODES, D_OUT), jnp.float32),
    )(p2, q2, y, dinv, W_edge.astype(jnp.float32), bias.reshape(1, D_OUT))
    return out


# speedup vs baseline: 13.7348x; 1.0859x over previous
import functools

import jax
import jax.numpy as jnp
from jax import lax
from jax.experimental import pallas as pl
from jax.experimental.pallas import tpu as pltpu
from jax.experimental.pallas import tpu_sc as plsc

N_NODES = 10000
N_EDGES = 320000
D_IN = 128
D_OUT = 128
D_EDGE = 16

NC = 2
NS = 16
NW = NC * NS
E_PER_W = N_EDGES // NW
CHUNK = 80
N_CHUNKS = E_PER_W // CHUNK
N_PAD = 10112
CA = 128
NFULL = E_PER_W // CA
TAIL = E_PER_W - NFULL * CA
ROWS_PER_TILE = N_PAD // NS

_sc_mesh = plsc.VectorSubcoreMesh(core_axis_name="c", subcore_axis_name="s")


@functools.partial(
    pl.kernel,
    out_type=jax.ShapeDtypeStruct((NC * N_PAD, D_OUT), jnp.float32),
    mesh=_sc_mesh,
    scratch_types=[
        pltpu.VMEM((2, CA), jnp.int32),
        pltpu.VMEM((TAIL,), jnp.int32),
        pltpu.VMEM((CA, D_OUT), jnp.float32),
        pltpu.VMEM((TAIL, D_OUT), jnp.float32),
        pltpu.SemaphoreType.DMA,
        pltpu.SemaphoreType.DMA,
        pltpu.VMEM_SHARED((N_PAD, D_OUT), jnp.float32),
    ],
)
def _deg_kernel(col_hbm, ones_hbm, z_hbm, deg_hbm, colv2, colt, onesv,
                onest, sem_i, sem_s, deg_sh):
    core = lax.axis_index("c")
    sid = lax.axis_index("s")
    wid = sid * NC + core
    r0 = sid * ROWS_PER_TILE

    pltpu.sync_copy(z_hbm, deg_sh.at[pl.ds(r0, ROWS_PER_TILE)])
    pltpu.sync_copy(ones_hbm, onesv)
    pltpu.sync_copy(ones_hbm.at[pl.ds(0, TAIL)], onest)
    plsc.subcore_barrier()

    base_w = wid * E_PER_W
    pltpu.sync_copy(col_hbm.at[pl.ds(base_w + NFULL * CA, TAIL)], colt)
    pltpu.sync_copy(onest, deg_sh.at[colt], add=True)

    def body(j, _):
        base = base_w + 2 * j * CA
        dca = pltpu.async_copy(col_hbm.at[pl.ds(base, CA)],
                               colv2.at[0], sem_i)
        dcb = pltpu.async_copy(col_hbm.at[pl.ds(base + CA, CA)],
                               colv2.at[1], sem_i)
        dca.wait()
        dcb.wait()
        d0 = pltpu.async_copy(onesv, deg_sh.at[colv2.at[0]], sem_s, add=True)
        d0.wait()
        d1 = pltpu.async_copy(onesv, deg_sh.at[colv2.at[1]], sem_s, add=True)
        d1.wait()
        return ()

    lax.fori_loop(0, NFULL // 2, body, ())
    plsc.subcore_barrier()
    pltpu.sync_copy(deg_sh.at[pl.ds(r0, ROWS_PER_TILE)],
                    deg_hbm.at[pl.ds(core * N_PAD + r0, ROWS_PER_TILE)])


def _scale_matmul_body(x_ref, w_ref, d0_ref, d1_ref, y_ref, dinv_ref):
    xt = jnp.dot(x_ref[...], w_ref[...], preferred_element_type=jnp.float32)
    d = d0_ref[:, 0:1] + d1_ref[:, 0:1] + 1.0
    dv = lax.rsqrt(d)
    dinv_ref[...] = dv
    y_ref[...] = xt * dv


@functools.partial(
    pl.kernel,
    out_type=jax.ShapeDtypeStruct((NC * N_PAD, D_OUT), jnp.float32),
    mesh=_sc_mesh,
    scratch_types=[
        pltpu.VMEM((2 * CHUNK,), jnp.int32),
        pltpu.VMEM((2, CHUNK), jnp.int32),
        pltpu.VMEM((CHUNK, D_OUT), jnp.float32),
        pltpu.VMEM((CHUNK, D_OUT), jnp.float32),
        pltpu.SemaphoreType.DMA,
        pltpu.SemaphoreType.DMA,
        pltpu.VMEM_SHARED((N_PAD, D_OUT), jnp.float32),
    ],
)
def _node_agg_kernel(row_hbm, col_hbm, y_hbm, zp_hbm, p_hbm,
                     rowv2, colv2, g0, g1, sem_i, sem_g, p_sh):
    core = lax.axis_index("c")
    sid = lax.axis_index("s")
    wid = sid * NC + core
    r0 = sid * ROWS_PER_TILE

    pltpu.sync_copy(zp_hbm, p_sh.at[pl.ds(r0, ROWS_PER_TILE)])
    plsc.subcore_barrier()

    base_w = wid * E_PER_W

    pltpu.sync_copy(row_hbm.at[pl.ds(base_w, CHUNK)],
                    rowv2.at[pl.ds(0, CHUNK)])
    pltpu.sync_copy(col_hbm.at[pl.ds(base_w, CHUNK)], colv2.at[0])
    pltpu.async_copy(y_hbm.at[rowv2.at[pl.ds(0, CHUNK)]], g0, sem_g).wait()
    pltpu.sync_copy(g0, p_sh.at[colv2.at[0]], add=True)

    def body(j, _):
        base = base_w + (1 + 2 * j) * CHUNK
        di = pltpu.async_copy(row_hbm.at[pl.ds(base, 2 * CHUNK)], rowv2,
                              sem_i)
        dca = pltpu.async_copy(col_hbm.at[pl.ds(base, CHUNK)],
                               colv2.at[0], sem_i)
        dcb = pltpu.async_copy(col_hbm.at[pl.ds(base + CHUNK, CHUNK)],
                               colv2.at[1], sem_i)
        di.wait()
        dca.wait()
        dcb.wait()
        dg0 = pltpu.async_copy(y_hbm.at[rowv2.at[pl.ds(0, CHUNK)]], g0,
                               sem_g)
        dg1 = pltpu.async_copy(y_hbm.at[rowv2.at[pl.ds(CHUNK, CHUNK)]], g1,
                               sem_g)
        dg0.wait()
        pltpu.sync_copy(g0, p_sh.at[colv2.at[0]], add=True)
        dg1.wait()
        pltpu.sync_copy(g1, p_sh.at[colv2.at[1]], add=True)
        return ()

    lax.fori_loop(0, (N_CHUNKS - 1) // 2, body, ())
    plsc.subcore_barrier()
    pltpu.sync_copy(p_sh.at[pl.ds(r0, ROWS_PER_TILE)],
                    p_hbm.at[pl.ds(core * N_PAD + r0, ROWS_PER_TILE)])


@functools.partial(
    pl.kernel,
    out_type=jax.ShapeDtypeStruct((NC * N_PAD, D_OUT), jnp.float32),
    mesh=_sc_mesh,
    scratch_types=[
        pltpu.VMEM((2 * CA,), jnp.int32),
        pltpu.VMEM((2, CA), jnp.int32),
        pltpu.VMEM((TAIL,), jnp.int32),
        pltpu.VMEM((CA, D_EDGE), jnp.float32),
        pltpu.VMEM((N_NODES + 16,), jnp.float32),
        pltpu.VMEM((CA, D_OUT), jnp.float32),
        pltpu.VMEM((TAIL, D_OUT), jnp.float32),
        pltpu.SemaphoreType.DMA,
        pltpu.SemaphoreType.DMA,
        pltpu.VMEM_SHARED((N_PAD, D_OUT), jnp.float32),
    ],
)
def _edge_agg_kernel(row_hbm, col_hbm, ea_hbm, dinv_hbm, zq_hbm, q_hbm,
                     rowv2, colv2, colt, eabuf, dloc, m0, mt, sem_i, sem_s,
                     q_sh):
    core = lax.axis_index("c")
    sid = lax.axis_index("s")
    wid = sid * NC + core
    r0 = sid * ROWS_PER_TILE

    pltpu.sync_copy(zq_hbm, q_sh.at[pl.ds(r0, ROWS_PER_TILE)])
    pltpu.sync_copy(zq_hbm.at[pl.ds(0, CA)], m0)
    pltpu.sync_copy(zq_hbm.at[pl.ds(0, TAIL)], mt)
    pltpu.sync_copy(dinv_hbm, dloc.at[pl.ds(0, N_NODES)])
    plsc.subcore_barrier()

    base_w = wid * E_PER_W

    def scale(off, nsteps, mbuf):
        for j in range(nsteps):
            rv = rowv2[pl.ds(off + j * 16, 16)]
            for i in range(16):
                e = j * 16 + i
                dv = dloc[pl.ds(rv[i], 16)]
                mbuf[e, pl.ds(0, D_EDGE)] = eabuf[e, :] * dv[0]

    tbase = base_w + NFULL * CA
    pltpu.sync_copy(row_hbm.at[pl.ds(tbase, TAIL)], rowv2.at[pl.ds(0, TAIL)])
    pltpu.sync_copy(col_hbm.at[pl.ds(tbase, TAIL)], colt)
    pltpu.sync_copy(ea_hbm.at[pl.ds(tbase, TAIL)], eabuf.at[pl.ds(0, TAIL)])
    scale(0, TAIL // 16, mt)
    pltpu.sync_copy(mt, q_sh.at[colt], add=True)

    def body(j, _):
        base = base_w + 2 * j * CA
        di = pltpu.async_copy(row_hbm.at[pl.ds(base, 2 * CA)], rowv2,
                              sem_i)
        dca = pltpu.async_copy(col_hbm.at[pl.ds(base, CA)],
                               colv2.at[0], sem_i)
        dcb = pltpu.async_copy(col_hbm.at[pl.ds(base + CA, CA)],
                               colv2.at[1], sem_i)
        de0 = pltpu.async_copy(ea_hbm.at[pl.ds(base, CA)], eabuf,
                               sem_i)
        di.wait()
        de0.wait()
        dca.wait()
        dcb.wait()
        scale(0, CA // 16, m0)
        d0 = pltpu.async_copy(m0, q_sh.at[colv2.at[0]], sem_s, add=True)
        de1 = pltpu.async_copy(ea_hbm.at[pl.ds(base + CA, CA)],
                               eabuf, sem_i)
        d0.wait()
        de1.wait()
        scale(CA, CA // 16, m0)
        pltpu.sync_copy(m0, q_sh.at[colv2.at[1]], add=True)
        return ()

    lax.fori_loop(0, NFULL // 2, body, ())
    plsc.subcore_barrier()
    pltpu.sync_copy(q_sh.at[pl.ds(r0, ROWS_PER_TILE)],
                    q_hbm.at[pl.ds(core * N_PAD + r0, ROWS_PER_TILE)])


def _combine_body(p_ref, q_ref, y_ref, dinv_ref, we_ref, b_ref, out_ref):
    q = q_ref[0][:, :D_EDGE] + q_ref[1][:, :D_EDGE]
    eat = jnp.dot(q, we_ref[...], preferred_element_type=jnp.float32)
    s = p_ref[0] + p_ref[1] + y_ref[...] + eat
    out_ref[...] = s * dinv_ref[...] + b_ref[...]


BLK = 400
N_BLKS = N_NODES // BLK


def kernel(x, edge_index, edge_attr, W_node, W_edge, bias):
    row = edge_index[0].astype(jnp.int32)
    col = edge_index[1].astype(jnp.int32)
    ea = edge_attr.astype(jnp.float32)

    ones_c = jnp.ones((CA, D_OUT), jnp.float32)
    z_rows = jnp.zeros((ROWS_PER_TILE, D_OUT), jnp.float32)

    deg2 = _deg_kernel(col, ones_c, z_rows)
    deg2 = deg2.reshape(NC, N_PAD, D_OUT)

    y, dinv = pl.pallas_call(
        _scale_matmul_body,
        grid=(N_BLKS,),
        in_specs=[
            pl.BlockSpec((BLK, D_IN), lambda i: (i, 0)),
            pl.BlockSpec((D_IN, D_OUT), lambda i: (0, 0)),
            pl.BlockSpec((BLK, D_OUT), lambda i: (i, 0)),
            pl.BlockSpec((BLK, D_OUT), lambda i: (i, 0)),
        ],
        out_specs=[
            pl.BlockSpec((BLK, D_OUT), lambda i: (i, 0)),
            pl.BlockSpec((BLK, 1), lambda i: (i, 0)),
        ],
        out_shape=[
            jax.ShapeDtypeStruct((N_NODES, D_OUT), jnp.float32),
            jax.ShapeDtypeStruct((N_NODES, 1), jnp.float32),
        ],
    )(x, W_node, deg2[0], deg2[1])

    p2 = _node_agg_kernel(row, col, y, z_rows).reshape(NC, N_PAD, D_OUT)
    q2 = _edge_agg_kernel(row, col, ea, dinv.reshape(N_NODES),
                          z_rows).reshape(NC, N_PAD, D_OUT)

    out = pl.pallas_call(
        _combine_body,
        grid=(N_BLKS,),
        in_specs=[
            pl.BlockSpec((NC, BLK, D_OUT), lambda i: (0, i, 0)),
            pl.BlockSpec((NC, BLK, D_OUT), lambda i: (0, i, 0)),
            pl.BlockSpec((BLK, D_OUT), lambda i: (i, 0)),
            pl.BlockSpec((BLK, 1), lambda i: (i, 0)),
            pl.BlockSpec((D_EDGE, D_OUT), lambda i: (0, 0)),
            pl.BlockSpec((1, D_OUT), lambda i: (0, 0)),
        ],
        out_specs=pl.BlockSpec((BLK, D_OUT), lambda i: (i, 0)),
        out_shape=jax.ShapeDtypeStruct((N_NODES, D_OUT), jnp.float32),
    )(p2, q2, y, dinv, W_edge.astype(jnp.float32), bias.reshape(1, D_OUT))
    return out


# speedup vs baseline: 14.0502x; 1.0230x over previous
import functools

import jax
import jax.numpy as jnp
from jax import lax
from jax.experimental import pallas as pl
from jax.experimental.pallas import tpu as pltpu
from jax.experimental.pallas import tpu_sc as plsc

N_NODES = 10000
N_EDGES = 320000
D_IN = 128
D_OUT = 128
D_EDGE = 16

NC = 2
NS = 16
NW = NC * NS
E_PER_W = N_EDGES // NW
CHUNK = 80
N_CHUNKS = E_PER_W // CHUNK
N_PAD = 10112
CA = 128
NFULL = E_PER_W // CA
TAIL = E_PER_W - NFULL * CA
ROWS_PER_TILE = N_PAD // NS

_sc_mesh = plsc.VectorSubcoreMesh(core_axis_name="c", subcore_axis_name="s")


@functools.partial(
    pl.kernel,
    out_type=jax.ShapeDtypeStruct((NC * N_PAD, D_OUT), jnp.float32),
    mesh=_sc_mesh,
    scratch_types=[
        pltpu.VMEM((2, CA), jnp.int32),
        pltpu.VMEM((TAIL,), jnp.int32),
        pltpu.VMEM((CA, D_OUT), jnp.float32),
        pltpu.VMEM((TAIL, D_OUT), jnp.float32),
        pltpu.SemaphoreType.DMA,
        pltpu.SemaphoreType.DMA,
        pltpu.VMEM_SHARED((N_PAD, D_OUT), jnp.float32),
    ],
)
def _deg_kernel(col_hbm, ones_hbm, z_hbm, deg_hbm, colv2, colt, onesv,
                onest, sem_i, sem_s, deg_sh):
    core = lax.axis_index("c")
    sid = lax.axis_index("s")
    wid = sid * NC + core
    r0 = sid * ROWS_PER_TILE

    pltpu.sync_copy(z_hbm, deg_sh.at[pl.ds(r0, ROWS_PER_TILE)])
    pltpu.sync_copy(ones_hbm, onesv)
    pltpu.sync_copy(ones_hbm.at[pl.ds(0, TAIL)], onest)
    plsc.subcore_barrier()

    base_w = wid * E_PER_W
    pltpu.sync_copy(col_hbm.at[pl.ds(base_w + NFULL * CA, TAIL)], colt)
    pltpu.sync_copy(onest, deg_sh.at[colt], add=True)

    def body(j, _):
        base = base_w + 2 * j * CA
        dca = pltpu.async_copy(col_hbm.at[pl.ds(base, CA)],
                               colv2.at[0], sem_i)
        dcb = pltpu.async_copy(col_hbm.at[pl.ds(base + CA, CA)],
                               colv2.at[1], sem_i)
        dca.wait()
        dcb.wait()
        d0 = pltpu.async_copy(onesv, deg_sh.at[colv2.at[0]], sem_s, add=True)
        d0.wait()
        d1 = pltpu.async_copy(onesv, deg_sh.at[colv2.at[1]], sem_s, add=True)
        d1.wait()
        return ()

    lax.fori_loop(0, NFULL // 2, body, ())
    plsc.subcore_barrier()
    pltpu.sync_copy(deg_sh.at[pl.ds(r0, ROWS_PER_TILE)],
                    deg_hbm.at[pl.ds(core * N_PAD + r0, ROWS_PER_TILE)])


def _scale_matmul_body(x_ref, w_ref, d0_ref, d1_ref, y_ref, dinv_ref):
    xt = jnp.dot(x_ref[...], w_ref[...], preferred_element_type=jnp.float32)
    d = d0_ref[:, 0:1] + d1_ref[:, 0:1] + 1.0
    dv = lax.rsqrt(d)
    dinv_ref[...] = dv
    y_ref[...] = xt * dv


CB = 96
NFULL_B = E_PER_W // CB
TAIL_B = E_PER_W - NFULL_B * CB


@functools.partial(
    pl.kernel,
    out_type=jax.ShapeDtypeStruct((NC * N_PAD, D_OUT), jnp.float32),
    mesh=_sc_mesh,
    scratch_types=[
        pltpu.VMEM((2 * CB,), jnp.int32),
        pltpu.VMEM((2, CB), jnp.int32),
        pltpu.VMEM((TAIL_B,), jnp.int32),
        pltpu.VMEM((TAIL_B,), jnp.int32),
        pltpu.VMEM((CB, D_OUT), jnp.float32),
        pltpu.VMEM((CB, D_OUT), jnp.float32),
        pltpu.VMEM((TAIL_B, D_OUT), jnp.float32),
        pltpu.SemaphoreType.DMA,
        pltpu.SemaphoreType.DMA,
        pltpu.VMEM_SHARED((N_PAD, D_OUT), jnp.float32),
    ],
)
def _node_agg_kernel(row_hbm, col_hbm, y_hbm, zp_hbm, p_hbm,
                     rowv2, colv2, rowt, colt, g0, g1, gt, sem_i, sem_g,
                     p_sh):
    core = lax.axis_index("c")
    sid = lax.axis_index("s")
    wid = sid * NC + core
    r0 = sid * ROWS_PER_TILE

    pltpu.sync_copy(zp_hbm, p_sh.at[pl.ds(r0, ROWS_PER_TILE)])
    plsc.subcore_barrier()

    base_w = wid * E_PER_W

    tbase = base_w + NFULL_B * CB
    pltpu.sync_copy(row_hbm.at[pl.ds(tbase, TAIL_B)], rowt)
    pltpu.sync_copy(col_hbm.at[pl.ds(tbase, TAIL_B)], colt)
    pltpu.async_copy(y_hbm.at[rowt], gt, sem_g).wait()
    pltpu.sync_copy(gt, p_sh.at[colt], add=True)

    def body(j, _):
        base = base_w + 2 * j * CB
        di = pltpu.async_copy(row_hbm.at[pl.ds(base, 2 * CB)], rowv2,
                              sem_i)
        dca = pltpu.async_copy(col_hbm.at[pl.ds(base, CB)],
                               colv2.at[0], sem_i)
        dcb = pltpu.async_copy(col_hbm.at[pl.ds(base + CB, CB)],
                               colv2.at[1], sem_i)
        di.wait()
        dca.wait()
        dcb.wait()
        dg0 = pltpu.async_copy(y_hbm.at[rowv2.at[pl.ds(0, CB)]], g0,
                               sem_g)
        dg1 = pltpu.async_copy(y_hbm.at[rowv2.at[pl.ds(CB, CB)]], g1,
                               sem_g)
        dg0.wait()
        pltpu.sync_copy(g0, p_sh.at[colv2.at[0]], add=True)
        dg1.wait()
        pltpu.sync_copy(g1, p_sh.at[colv2.at[1]], add=True)
        return ()

    lax.fori_loop(0, NFULL_B // 2, body, ())
    plsc.subcore_barrier()
    pltpu.sync_copy(p_sh.at[pl.ds(r0, ROWS_PER_TILE)],
                    p_hbm.at[pl.ds(core * N_PAD + r0, ROWS_PER_TILE)])


@functools.partial(
    pl.kernel,
    out_type=jax.ShapeDtypeStruct((NC * N_PAD, D_OUT), jnp.float32),
    mesh=_sc_mesh,
    scratch_types=[
        pltpu.VMEM((2 * CA,), jnp.int32),
        pltpu.VMEM((2, CA), jnp.int32),
        pltpu.VMEM((TAIL,), jnp.int32),
        pltpu.VMEM((CA, D_EDGE), jnp.float32),
        pltpu.VMEM((N_NODES + 16,), jnp.float32),
        pltpu.VMEM((CA, D_OUT), jnp.float32),
        pltpu.VMEM((TAIL, D_OUT), jnp.float32),
        pltpu.SemaphoreType.DMA,
        pltpu.SemaphoreType.DMA,
        pltpu.VMEM_SHARED((N_PAD, D_OUT), jnp.float32),
    ],
)
def _edge_agg_kernel(row_hbm, col_hbm, ea_hbm, dinv_hbm, zq_hbm, q_hbm,
                     rowv2, colv2, colt, eabuf, dloc, m0, mt, sem_i, sem_s,
                     q_sh):
    core = lax.axis_index("c")
    sid = lax.axis_index("s")
    wid = sid * NC + core
    r0 = sid * ROWS_PER_TILE

    pltpu.sync_copy(zq_hbm, q_sh.at[pl.ds(r0, ROWS_PER_TILE)])
    pltpu.sync_copy(zq_hbm.at[pl.ds(0, CA)], m0)
    pltpu.sync_copy(zq_hbm.at[pl.ds(0, TAIL)], mt)
    pltpu.sync_copy(dinv_hbm, dloc.at[pl.ds(0, N_NODES)])
    plsc.subcore_barrier()

    base_w = wid * E_PER_W

    def scale(off, nsteps, mbuf):
        for j in range(nsteps):
            rv = rowv2[pl.ds(off + j * 16, 16)]
            for i in range(16):
                e = j * 16 + i
                dv = dloc[pl.ds(rv[i], 16)]
                mbuf[e, pl.ds(0, D_EDGE)] = eabuf[e, :] * dv[0]

    tbase = base_w + NFULL * CA
    pltpu.sync_copy(row_hbm.at[pl.ds(tbase, TAIL)], rowv2.at[pl.ds(0, TAIL)])
    pltpu.sync_copy(col_hbm.at[pl.ds(tbase, TAIL)], colt)
    pltpu.sync_copy(ea_hbm.at[pl.ds(tbase, TAIL)], eabuf.at[pl.ds(0, TAIL)])
    scale(0, TAIL // 16, mt)
    pltpu.sync_copy(mt, q_sh.at[colt], add=True)

    def body(j, _):
        base = base_w + 2 * j * CA
        di = pltpu.async_copy(row_hbm.at[pl.ds(base, 2 * CA)], rowv2,
                              sem_i)
        dca = pltpu.async_copy(col_hbm.at[pl.ds(base, CA)],
                               colv2.at[0], sem_i)
        dcb = pltpu.async_copy(col_hbm.at[pl.ds(base + CA, CA)],
                               colv2.at[1], sem_i)
        de0 = pltpu.async_copy(ea_hbm.at[pl.ds(base, CA)], eabuf,
                               sem_i)
        di.wait()
        de0.wait()
        dca.wait()
        dcb.wait()
        scale(0, CA // 16, m0)
        d0 = pltpu.async_copy(m0, q_sh.at[colv2.at[0]], sem_s, add=True)
        de1 = pltpu.async_copy(ea_hbm.at[pl.ds(base + CA, CA)],
                               eabuf, sem_i)
        d0.wait()
        de1.wait()
        scale(CA, CA // 16, m0)
        pltpu.sync_copy(m0, q_sh.at[colv2.at[1]], add=True)
        return ()

    lax.fori_loop(0, NFULL // 2, body, ())
    plsc.subcore_barrier()
    pltpu.sync_copy(q_sh.at[pl.ds(r0, ROWS_PER_TILE)],
                    q_hbm.at[pl.ds(core * N_PAD + r0, ROWS_PER_TILE)])


def _combine_body(p_ref, q_ref, y_ref, dinv_ref, we_ref, b_ref, out_ref):
    q = q_ref[0][:, :D_EDGE] + q_ref[1][:, :D_EDGE]
    eat = jnp.dot(q, we_ref[...], preferred_element_type=jnp.float32)
    s = p_ref[0] + p_ref[1] + y_ref[...] + eat
    out_ref[...] = s * dinv_ref[...] + b_ref[...]


BLK = 400
N_BLKS = N_NODES // BLK


def kernel(x, edge_index, edge_attr, W_node, W_edge, bias):
    row = edge_index[0].astype(jnp.int32)
    col = edge_index[1].astype(jnp.int32)
    ea = edge_attr.astype(jnp.float32)

    ones_c = jnp.ones((CA, D_OUT), jnp.float32)
    z_rows = jnp.zeros((ROWS_PER_TILE, D_OUT), jnp.float32)

    deg2 = _deg_kernel(col, ones_c, z_rows)
    deg2 = deg2.reshape(NC, N_PAD, D_OUT)

    y, dinv = pl.pallas_call(
        _scale_matmul_body,
        grid=(N_BLKS,),
        in_specs=[
            pl.BlockSpec((BLK, D_IN), lambda i: (i, 0)),
            pl.BlockSpec((D_IN, D_OUT), lambda i: (0, 0)),
            pl.BlockSpec((BLK, D_OUT), lambda i: (i, 0)),
            pl.BlockSpec((BLK, D_OUT), lambda i: (i, 0)),
        ],
        out_specs=[
            pl.BlockSpec((BLK, D_OUT), lambda i: (i, 0)),
            pl.BlockSpec((BLK, 1), lambda i: (i, 0)),
        ],
        out_shape=[
            jax.ShapeDtypeStruct((N_NODES, D_OUT), jnp.float32),
            jax.ShapeDtypeStruct((N_NODES, 1), jnp.float32),
        ],
    )(x, W_node, deg2[0], deg2[1])

    p2 = _node_agg_kernel(row, col, y, z_rows).reshape(NC, N_PAD, D_OUT)
    q2 = _edge_agg_kernel(row, col, ea, dinv.reshape(N_NODES),
                          z_rows).reshape(NC, N_PAD, D_OUT)

    out = pl.pallas_call(
        _combine_body,
        grid=(N_BLKS,),
        in_specs=[
            pl.BlockSpec((NC, BLK, D_OUT), lambda i: (0, i, 0)),
            pl.BlockSpec((NC, BLK, D_OUT), lambda i: (0, i, 0)),
            pl.BlockSpec((BLK, D_OUT), lambda i: (i, 0)),
            pl.BlockSpec((BLK, 1), lambda i: (i, 0)),
            pl.BlockSpec((D_EDGE, D_OUT), lambda i: (0, 0)),
            pl.BlockSpec((1, D_OUT), lambda i: (0, 0)),
        ],
        out_specs=pl.BlockSpec((BLK, D_OUT), lambda i: (i, 0)),
        out_shape=jax.ShapeDtypeStruct((N_NODES, D_OUT), jnp.float32),
    )(p2, q2, y, dinv, W_edge.astype(jnp.float32), bias.reshape(1, D_OUT))
    return out
